# baseline (speedup 1.0000x reference)
import functools

import jax
import jax.numpy as jnp
import numpy as _np
from jax import lax
from jax.experimental import pallas as pl
from jax.experimental.pallas import tpu as pltpu
from jax.experimental.pallas import tpu_sc as plsc

_B = 32
_D = 64
_H = 32
_W = 32
_T = _B * _H * _W
_NCB = 8
_DPL = 8
_L = 16
_RPW = _H * _W
_CH = 128

_G = _np.linspace(-1.0, 1.0, 8).astype(_np.float32)
_THR = _np.arctanh(((_G[:-1].astype(_np.float64)
                     + _G[1:].astype(_np.float64)) / 2.0)).astype(_np.float32)
_C0, _C1, _C2, _C3, _C4, _C5, _C6 = (float(t) for t in _THR)

_PACKW = _np.zeros((_D, _NCB), _np.float32)
for _d in range(_D):
    _PACKW[_d, _d // _DPL] = float(_DPL ** (_DPL - 1 - _d % _DPL))


def _bucket(x):
    b2 = x > _C3
    t1 = jnp.where(b2, jnp.float32(_C5), jnp.float32(_C1))
    b1 = x > t1
    hi = jnp.where(b1, jnp.float32(_C6), jnp.float32(_C4))
    lo = jnp.where(b1, jnp.float32(_C2), jnp.float32(_C0))
    b0 = x > jnp.where(b2, hi, lo)
    return (jnp.where(b2, jnp.float32(4.0), jnp.float32(0.0))
            + jnp.where(b1, jnp.float32(2.0), jnp.float32(0.0))
            + jnp.where(b0, jnp.float32(1.0), jnp.float32(0.0)))


_mesh = plsc.VectorSubcoreMesh(core_axis_name="c", subcore_axis_name="s")


_NCHUNK = _RPW // _CH


@functools.partial(
    pl.kernel,
    out_type=jax.ShapeDtypeStruct((_T, _D), jnp.float32),
    mesh=_mesh,
    scratch_types=[
        pltpu.VMEM((_CH, _D), jnp.float32),
        pltpu.VMEM((_CH, _D), jnp.float32),
        pltpu.VMEM((_CH, _D), jnp.float32),
        pltpu.VMEM((_CH, _D), jnp.float32),
        pltpu.SemaphoreType.DMA,
        pltpu.SemaphoreType.DMA,
        pltpu.SemaphoreType.DMA,
        pltpu.SemaphoreType.DMA,
    ],
)
def _fsq_q_sc(z_hbm, q_hbm, in0, in1, q0, q1, si0, si1, so0, so1):
    b = lax.axis_index("s") * 2 + lax.axis_index("c")
    base = b * _RPW
    ins, qs = (in0, in1), (q0, q1)
    isems, osems = (si0, si1), (so0, so1)

    def rows(c):
        return z_hbm.at[pl.ds(base + c * _CH, _CH)]

    def orows(c):
        return q_hbm.at[pl.ds(base + c * _CH, _CH)]

    in_flight = [
        pltpu.async_copy(rows(0), in0, si0),
        pltpu.async_copy(rows(1), in1, si1),
    ]
    out_flight = [None, None]

    for c in range(_NCHUNK):
        s = c % 2
        ib, qb = ins[s], qs[s]
        in_flight[s].wait()
        if out_flight[s] is not None:
            out_flight[s].wait()

        def body(j, carry):
            for k in range(4):
                r = j * 4 + k
                for v in range(_D // _L):
                    off = pl.multiple_of(v * _L, _L)
                    idxf = _bucket(ib[r, pl.ds(off, _L)])
                    qb[r, pl.ds(off, _L)] = (
                        idxf * jnp.float32(2.0 / 7.0) - 1.0
                    )
            return carry

        lax.fori_loop(0, _CH // 4, body, 0)

        out_flight[s] = pltpu.async_copy(qb, orows(c), osems[s])
        if c + 2 < _NCHUNK:
            in_flight[s] = pltpu.async_copy(rows(c + 2), ib, isems[s])

    out_flight[0].wait()
    out_flight[1].wait()


def _idx_tc_body(w_ref, z_ref, out_ref):
    zb = z_ref[...]
    digits = _bucket(zb)
    packed = jax.lax.dot(
        digits, w_ref[...],
        preferred_element_type=jnp.float32,
    )
    packed = packed.astype(jnp.int32)
    out_ref[...] = packed.T.reshape(1, _NCB, _H // 2, _W)


_TBLK = 512


_idx_tc = pl.pallas_call(
    _idx_tc_body,
    grid=(_T // _TBLK,),
    in_specs=[pl.BlockSpec((_D, _NCB), lambda i: (0, 0)),
              pl.BlockSpec((_TBLK, _D), lambda i: (i, 0),
                           )],
    out_specs=pl.BlockSpec((1, _NCB, _H // 2, _W),
                           lambda i: (i // 2, 0, i % 2, 0)),
    out_shape=jax.ShapeDtypeStruct((_B, _NCB, _H, _W), jnp.int32),
)


def kernel(z):
    B, D, H, W = z.shape
    zt = jnp.transpose(z, (0, 2, 3, 1)).reshape(_T, _D)
    q = _fsq_q_sc(zt)
    idx = _idx_tc(jnp.asarray(_PACKW), zt)
    qt = jnp.transpose(q.reshape(B, H, W, D), (0, 3, 1, 2))
    return qt, idx


# speedup vs baseline: 1.3675x; 1.3675x over previous
import functools

import jax
import jax.numpy as jnp
import numpy as _np
from jax import lax
from jax.experimental import pallas as pl
from jax.experimental.pallas import tpu as pltpu
from jax.experimental.pallas import tpu_sc as plsc

_B = 32
_D = 64
_H = 32
_W = 32
_T = _B * _H * _W
_NCB = 8
_DPL = 8
_L = 16
_RPW = _H * _W
_CH = 128

_G = _np.linspace(-1.0, 1.0, 8).astype(_np.float32)
_THR = _np.arctanh(((_G[:-1].astype(_np.float64)
                     + _G[1:].astype(_np.float64)) / 2.0)).astype(_np.float32)
_C0, _C1, _C2, _C3, _C4, _C5, _C6 = (float(t) for t in _THR)

_PACKW = _np.zeros((_D, _NCB), _np.float32)
for _d in range(_D):
    _PACKW[_d, _d // _DPL] = float(_DPL ** (_DPL - 1 - _d % _DPL))


def _bucket(x):
    b2 = x > _C3
    t1 = jnp.where(b2, jnp.float32(_C5), jnp.float32(_C1))
    b1 = x > t1
    hi = jnp.where(b1, jnp.float32(_C6), jnp.float32(_C4))
    lo = jnp.where(b1, jnp.float32(_C2), jnp.float32(_C0))
    b0 = x > jnp.where(b2, hi, lo)
    return (jnp.where(b2, jnp.float32(4.0), jnp.float32(0.0))
            + jnp.where(b1, jnp.float32(2.0), jnp.float32(0.0))
            + jnp.where(b0, jnp.float32(1.0), jnp.float32(0.0)))


_mesh = plsc.VectorSubcoreMesh(core_axis_name="c", subcore_axis_name="s")


_NCHUNK = _RPW // _CH


@functools.partial(
    pl.kernel,
    out_type=jax.ShapeDtypeStruct((_T, _D), jnp.float32),
    mesh=_mesh,
    scratch_types=[
        pltpu.VMEM((_CH, _D), jnp.float32),
        pltpu.VMEM((_CH, _D), jnp.float32),
        pltpu.VMEM((_CH, _D), jnp.float32),
        pltpu.VMEM((_CH, _D), jnp.float32),
        pltpu.SemaphoreType.DMA,
        pltpu.SemaphoreType.DMA,
        pltpu.SemaphoreType.DMA,
        pltpu.SemaphoreType.DMA,
    ],
)
def _fsq_q_sc(z_hbm, q_hbm, in0, in1, q0, q1, si0, si1, so0, so1):
    b = lax.axis_index("s") * 2 + lax.axis_index("c")
    base = b * _RPW

    def rows(c):
        return z_hbm.at[pl.ds(base + c * _CH, _CH)]

    def orows(c):
        return q_hbm.at[pl.ds(base + c * _CH, _CH)]

    def compute(ib, qb):
        def body(j, carry):
            for k in range(4):
                r = j * 4 + k
                for v in range(_D // _L):
                    off = pl.multiple_of(v * _L, _L)
                    idxf = _bucket(ib[r, pl.ds(off, _L)])
                    qb[r, pl.ds(off, _L)] = (
                        idxf * jnp.float32(2.0 / 7.0) - 1.0
                    )
            return carry

        lax.fori_loop(0, _CH // 4, body, 0)

    pltpu.async_copy(rows(0), in0, si0)
    pltpu.async_copy(rows(1), in1, si1)

    def pair(c2, carry):
        c = c2 * 2
        for s, ib, qb, isem, osem in (
            (0, in0, q0, si0, so0),
            (1, in1, q1, si1, so1),
        ):
            pltpu.make_async_copy(rows(c + s), ib, isem).wait()

            @pl.when(c2 > 0)
            def _():
                pltpu.make_async_copy(qb, orows(c + s - 2), osem).wait()

            compute(ib, qb)
            pltpu.async_copy(qb, orows(c + s), osem)

            @pl.when(c2 < _NCHUNK // 2 - 1)
            def _():
                pltpu.async_copy(rows(c + s + 2), ib, isem)
        return carry

    lax.fori_loop(0, _NCHUNK // 2, pair, 0)

    pltpu.make_async_copy(q0, orows(_NCHUNK - 2), so0).wait()
    pltpu.make_async_copy(q1, orows(_NCHUNK - 1), so1).wait()


def _idx_tc_body(w_ref, z_ref, out_ref):
    zb = z_ref[...]
    digits = _bucket(zb)
    packed = jax.lax.dot(
        digits, w_ref[...],
        preferred_element_type=jnp.float32,
    )
    packed = packed.astype(jnp.int32)
    out_ref[...] = packed.T.reshape(1, _NCB, _H, _W)


_idx_tc = pl.pallas_call(
    _idx_tc_body,
    grid=(_B,),
    in_specs=[pl.BlockSpec((_D, _NCB), lambda b: (0, 0)),
              pl.BlockSpec((_RPW, _D), lambda b: (b, 0))],
    out_specs=pl.BlockSpec((1, _NCB, _H, _W), lambda b: (b, 0, 0, 0)),
    out_shape=jax.ShapeDtypeStruct((_B, _NCB, _H, _W), jnp.int32),
)


def kernel(z):
    B, D, H, W = z.shape
    zt = jnp.transpose(z, (0, 2, 3, 1)).reshape(_T, _D)
    q = _fsq_q_sc(zt)
    zt_hbm = pltpu.with_memory_space_constraint(zt, pltpu.MemorySpace.HBM)
    w_hbm = pltpu.with_memory_space_constraint(
        jnp.asarray(_PACKW), pltpu.MemorySpace.HBM)
    idx = _idx_tc(w_hbm, zt_hbm)
    qt = jnp.transpose(q.reshape(B, H, W, D), (0, 3, 1, 2))
    return qt, idx


# speedup vs baseline: 1.6849x; 1.2321x over previous
import functools

import jax
import jax.numpy as jnp
import numpy as _np
from jax import lax
from jax.experimental import pallas as pl
from jax.experimental.pallas import tpu as pltpu
from jax.experimental.pallas import tpu_sc as plsc

_B = 32
_D = 64
_H = 32
_W = 32
_T = _B * _H * _W
_NCB = 8
_DPL = 8
_L = 16
_RPW = _H * _W
_CH = 128

_G = _np.linspace(-1.0, 1.0, 8).astype(_np.float32)
_THR = _np.arctanh(((_G[:-1].astype(_np.float64)
                     + _G[1:].astype(_np.float64)) / 2.0)).astype(_np.float32)
_C0, _C1, _C2, _C3, _C4, _C5, _C6 = (float(t) for t in _THR)

_PACKW = _np.zeros((_D, _NCB), _np.float32)
for _d in range(_D):
    _PACKW[_d, _d // _DPL] = float(_DPL ** (_DPL - 1 - _d % _DPL))


def _bucket(x):
    b2 = x > _C3
    t1 = jnp.where(b2, jnp.float32(_C5), jnp.float32(_C1))
    b1 = x > t1
    hi = jnp.where(b1, jnp.float32(_C6), jnp.float32(_C4))
    lo = jnp.where(b1, jnp.float32(_C2), jnp.float32(_C0))
    b0 = x > jnp.where(b2, hi, lo)
    return (jnp.where(b2, jnp.float32(4.0), jnp.float32(0.0))
            + jnp.where(b1, jnp.float32(2.0), jnp.float32(0.0))
            + jnp.where(b0, jnp.float32(1.0), jnp.float32(0.0)))


_mesh = plsc.VectorSubcoreMesh(core_axis_name="c", subcore_axis_name="s")


_NCHUNK = _RPW // _CH


@functools.partial(
    pl.kernel,
    out_type=jax.ShapeDtypeStruct((_T, _D), jnp.float32),
    mesh=_mesh,
    scratch_types=[
        pltpu.VMEM((_CH, _D), jnp.float32),
        pltpu.VMEM((_CH, _D), jnp.float32),
        pltpu.VMEM((_CH, _D), jnp.float32),
        pltpu.VMEM((_CH, _D), jnp.float32),
        pltpu.SemaphoreType.DMA,
        pltpu.SemaphoreType.DMA,
        pltpu.SemaphoreType.DMA,
        pltpu.SemaphoreType.DMA,
    ],
)
def _fsq_q_sc(z_hbm, q_hbm, in0, in1, q0, q1, si0, si1, so0, so1):
    b = lax.axis_index("s") * 2 + lax.axis_index("c")
    base = b * _RPW

    def rows(c):
        return z_hbm.at[pl.ds(base + c * _CH, _CH)]

    def orows(c):
        return q_hbm.at[pl.ds(base + c * _CH, _CH)]

    def compute(ib, qb):
        def body(j, carry):
            for k in range(4):
                r = j * 4 + k
                for v in range(_D // _L):
                    off = pl.multiple_of(v * _L, _L)
                    idxf = _bucket(ib[r, pl.ds(off, _L)])
                    qb[r, pl.ds(off, _L)] = (
                        idxf * jnp.float32(2.0 / 7.0) - 1.0
                    )
            return carry

        lax.fori_loop(0, _CH // 4, body, 0)

    pltpu.async_copy(rows(0), in0, si0)
    pltpu.async_copy(rows(1), in1, si1)

    def pair(c2, carry):
        c = c2 * 2
        for s, ib, qb, isem, osem in (
            (0, in0, q0, si0, so0),
            (1, in1, q1, si1, so1),
        ):
            pltpu.make_async_copy(rows(c + s), ib, isem).wait()

            @pl.when(c2 > 0)
            def _():
                pltpu.make_async_copy(qb, orows(c + s - 2), osem).wait()

            compute(ib, qb)
            pltpu.async_copy(qb, orows(c + s), osem)

            @pl.when(c2 < _NCHUNK // 2 - 1)
            def _():
                pltpu.async_copy(rows(c + s + 2), ib, isem)
        return carry

    lax.fori_loop(0, _NCHUNK // 2, pair, 0)

    pltpu.make_async_copy(q0, orows(_NCHUNK - 2), so0).wait()
    pltpu.make_async_copy(q1, orows(_NCHUNK - 1), so1).wait()


def _idx_tc_body(w_ref, z_ref, out_ref):
    zb = z_ref[...]
    digits = _bucket(zb)
    packed = jax.lax.dot(
        digits, w_ref[...],
        preferred_element_type=jnp.float32,
    )
    packed = packed.astype(jnp.int32)
    pb = packed.reshape(_BPB, _RPW, _NCB)
    out_ref[...] = jnp.transpose(pb, (0, 2, 1)).reshape(
        _BPB, _NCB, _H, _W)


_BPB = 4


_idx_tc = pl.pallas_call(
    _idx_tc_body,
    grid=(_B // _BPB,),
    in_specs=[pl.BlockSpec((_D, _NCB), lambda b: (0, 0)),
              pl.BlockSpec((_BPB * _RPW, _D), lambda b: (b, 0))],
    out_specs=pl.BlockSpec((_BPB, _NCB, _H, _W), lambda b: (b, 0, 0, 0)),
    out_shape=jax.ShapeDtypeStruct((_B, _NCB, _H, _W), jnp.int32),
)


def kernel(z):
    B, D, H, W = z.shape
    zt = jnp.transpose(z, (0, 2, 3, 1)).reshape(_T, _D)
    q = _fsq_q_sc(zt)
    zt_hbm = pltpu.with_memory_space_constraint(zt, pltpu.MemorySpace.HBM)
    w_hbm = pltpu.with_memory_space_constraint(
        jnp.asarray(_PACKW), pltpu.MemorySpace.HBM)
    idx = _idx_tc(w_hbm, zt_hbm)
    qt = jnp.transpose(q.reshape(B, H, W, D), (0, 3, 1, 2))
    return qt, idx


# speedup vs baseline: 1.7016x; 1.0099x over previous
import functools

import jax
import jax.numpy as jnp
import numpy as _np
from jax import lax
from jax.experimental import pallas as pl
from jax.experimental.pallas import tpu as pltpu
from jax.experimental.pallas import tpu_sc as plsc

_B = 32
_D = 64
_H = 32
_W = 32
_T = _B * _H * _W
_NCB = 8
_DPL = 8
_L = 16
_RPW = _H * _W
_CH = 128

_G = _np.linspace(-1.0, 1.0, 8).astype(_np.float32)
_THR = _np.arctanh(((_G[:-1].astype(_np.float64)
                     + _G[1:].astype(_np.float64)) / 2.0)).astype(_np.float32)
_C0, _C1, _C2, _C3, _C4, _C5, _C6 = (float(t) for t in _THR)

_PACKW = _np.zeros((_D, _NCB), _np.float32)
for _d in range(_D):
    _PACKW[_d, _d // _DPL] = float(_DPL ** (_DPL - 1 - _d % _DPL))


def _bucket(x):
    b2 = x > _C3
    t1 = jnp.where(b2, jnp.float32(_C5), jnp.float32(_C1))
    b1 = x > t1
    hi = jnp.where(b1, jnp.float32(_C6), jnp.float32(_C4))
    lo = jnp.where(b1, jnp.float32(_C2), jnp.float32(_C0))
    b0 = x > jnp.where(b2, hi, lo)
    return (jnp.where(b2, jnp.float32(4.0), jnp.float32(0.0))
            + jnp.where(b1, jnp.float32(2.0), jnp.float32(0.0))
            + jnp.where(b0, jnp.float32(1.0), jnp.float32(0.0)))


_mesh = plsc.VectorSubcoreMesh(core_axis_name="c", subcore_axis_name="s")


_NCHUNK = _RPW // _CH


@functools.partial(
    pl.kernel,
    out_type=jax.ShapeDtypeStruct((_T, _D), jnp.float32),
    mesh=_mesh,
    scratch_types=[
        pltpu.VMEM((_CH, _D), jnp.float32),
        pltpu.VMEM((_CH, _D), jnp.float32),
        pltpu.VMEM((_CH, _D), jnp.float32),
        pltpu.VMEM((_CH, _D), jnp.float32),
        pltpu.SemaphoreType.DMA,
        pltpu.SemaphoreType.DMA,
        pltpu.SemaphoreType.DMA,
        pltpu.SemaphoreType.DMA,
    ],
)
def _fsq_q_sc(z_hbm, q_hbm, in0, in1, q0, q1, si0, si1, so0, so1):
    b = lax.axis_index("s") * 2 + lax.axis_index("c")
    base = b * _RPW

    def rows(c):
        return z_hbm.at[pl.ds(base + c * _CH, _CH)]

    def orows(c):
        return q_hbm.at[pl.ds(base + c * _CH, _CH)]

    def compute(ib, qb):
        def body(j, carry):
            for k in range(4):
                r = j * 4 + k
                for v in range(_D // _L):
                    off = pl.multiple_of(v * _L, _L)
                    idxf = _bucket(ib[r, pl.ds(off, _L)])
                    qb[r, pl.ds(off, _L)] = (
                        idxf * jnp.float32(2.0 / 7.0) - 1.0
                    )
            return carry

        lax.fori_loop(0, _CH // 4, body, 0)

    pltpu.async_copy(rows(0), in0, si0)
    pltpu.async_copy(rows(1), in1, si1)

    def pair(c2, carry):
        c = c2 * 2
        for s, ib, qb, isem, osem in (
            (0, in0, q0, si0, so0),
            (1, in1, q1, si1, so1),
        ):
            pltpu.make_async_copy(rows(c + s), ib, isem).wait()

            @pl.when(c2 > 0)
            def _():
                pltpu.make_async_copy(qb, orows(c + s - 2), osem).wait()

            compute(ib, qb)
            pltpu.async_copy(qb, orows(c + s), osem)

            @pl.when(c2 < _NCHUNK // 2 - 1)
            def _():
                pltpu.async_copy(rows(c + s + 2), ib, isem)
        return carry

    lax.fori_loop(0, _NCHUNK // 2, pair, 0)

    pltpu.make_async_copy(q0, orows(_NCHUNK - 2), so0).wait()
    pltpu.make_async_copy(q1, orows(_NCHUNK - 1), so1).wait()


def _idx_tc_body(w_ref, z_ref, out_ref):
    zb = z_ref[...]
    digits = _bucket(zb)
    packed = jax.lax.dot(
        digits, w_ref[...],
        preferred_element_type=jnp.float32,
    )
    packed = packed.astype(jnp.int32)
    pb = packed.reshape(_BPB, _RPW, _NCB)
    out_ref[...] = jnp.transpose(pb, (0, 2, 1)).reshape(
        _BPB, _NCB, _H, _W)


_BPB = 8


_idx_tc = pl.pallas_call(
    _idx_tc_body,
    grid=(_B // _BPB,),
    in_specs=[pl.BlockSpec((_D, _NCB), lambda b: (0, 0)),
              pl.BlockSpec((_BPB * _RPW, _D), lambda b: (b, 0))],
    out_specs=pl.BlockSpec((_BPB, _NCB, _H, _W), lambda b: (b, 0, 0, 0)),
    out_shape=jax.ShapeDtypeStruct((_B, _NCB, _H, _W), jnp.int32),
)


def kernel(z):
    B, D, H, W = z.shape
    zt = jnp.transpose(z, (0, 2, 3, 1)).reshape(_T, _D)
    q = _fsq_q_sc(zt)
    zt_hbm = pltpu.with_memory_space_constraint(zt, pltpu.MemorySpace.HBM)
    w_hbm = pltpu.with_memory_space_constraint(
        jnp.asarray(_PACKW), pltpu.MemorySpace.HBM)
    idx = _idx_tc(w_hbm, zt_hbm)
    qt = jnp.transpose(q.reshape(B, H, W, D), (0, 3, 1, 2))
    return qt, idx
